# baseline (speedup 1.0000x reference)
import functools

import jax
import jax.numpy as jnp
from jax import lax
from jax.experimental import pallas as pl
from jax.experimental.pallas import tpu as pltpu
from jax.experimental.pallas import tpu_sc as plsc

_LANES = 16


def _bcast(v, t):
    dn = lax.GatherDimensionNumbers(
        offset_dims=(), collapsed_slice_dims=(0,), start_index_map=(0,))
    return lax.gather(v, jnp.full((_LANES, 1), t, jnp.int32), dn, (1,),
                      mode=lax.GatherScatterMode.PROMISE_IN_BOUNDS)


def _stage_a(cur_msg, WqT, Wk, bq2):
    B, D = cur_msg.shape
    BM = 256
    scale = D ** -0.5

    def body(msg_ref, WqT_ref, Wk_ref, bq2_ref, q2_ref, Wqk_s, bqk_s):
        @pl.when(pl.program_id(0) == 0)
        def _():
            Wqk_s[...] = jnp.dot(WqT_ref[...], Wk_ref[...],
                                 preferred_element_type=jnp.float32) * scale
            bqk_s[...] = jnp.dot(bq2_ref[...], Wk_ref[...],
                                 preferred_element_type=jnp.float32) * scale

        q2_ref[...] = jnp.dot(msg_ref[...].astype(jnp.bfloat16),
                              Wqk_s[...].astype(jnp.bfloat16),
                              preferred_element_type=jnp.float32) + bqk_s[...]

    return pl.pallas_call(
        body,
        grid=(B // BM,),
        in_specs=[
            pl.BlockSpec((BM, D), lambda i: (i, 0)),
            pl.BlockSpec((D, D), lambda i: (0, 0)),
            pl.BlockSpec((D, D), lambda i: (0, 0)),
            pl.BlockSpec((1, D), lambda i: (0, 0)),
        ],
        out_specs=pl.BlockSpec((BM, D), lambda i: (i, 0)),
        out_shape=jax.ShapeDtypeStruct((B, D), jnp.float32),
        scratch_shapes=[pltpu.VMEM((D, D), jnp.float32),
                        pltpu.VMEM((1, D), jnp.float32)],
    )(cur_msg, WqT, Wk, bq2)


def _bcast_dyn(v, t):
    dn = lax.GatherDimensionNumbers(
        offset_dims=(), collapsed_slice_dims=(0,), start_index_map=(0,))
    return lax.gather(v, jnp.full((_LANES, 1), 1, jnp.int32) * t, dn, (1,),
                      mode=lax.GatherScatterMode.PROMISE_IN_BOUNDS)


def _shuf(v, ix):
    dn = lax.GatherDimensionNumbers(
        offset_dims=(), collapsed_slice_dims=(0,), start_index_map=(0,))
    return lax.gather(v, ix.reshape(_LANES, 1), dn, (1,),
                      mode=lax.GatherScatterMode.PROMISE_IN_BOUNDS)


def _stage_b(idx, q2, bank3, M, b_sc):
    B, D = q2.shape
    NW = 32
    bw = b_sc // NW
    CB = 4
    nch = bw // CB
    DC = D // _LANES

    mesh = plsc.VectorSubcoreMesh(core_axis_name="c", subcore_axis_name="s")

    @functools.partial(
        pl.kernel, mesh=mesh,
        out_type=jax.ShapeDtypeStruct((b_sc, D), jnp.float32),
        compiler_params=pltpu.CompilerParams(needs_layout_passes=False),
        scratch_types=[
            pltpu.VMEM((bw,), jnp.int32),
            pltpu.VMEM((2, CB * M), jnp.int32),
            pltpu.VMEM((2, CB * M, D), jnp.float32),
            pltpu.VMEM((2, CB, D), jnp.float32),
            pltpu.VMEM((2, CB, D), jnp.float32),
            pltpu.SemaphoreType.DMA,
            pltpu.SemaphoreType.DMA,
        ])
    def sc(idx_hbm, q2_hbm, bank_hbm, p_hbm,
           idx_v, isub_v, rows_v, q2_v, p_v, sem0, sem1):
        wid = lax.axis_index("s") * 2 + lax.axis_index("c")
        base = wid * bw
        pltpu.sync_copy(idx_hbm.at[pl.ds(base, bw)], idx_v)
        lane = lax.broadcasted_iota(jnp.int32, (_LANES,), 0)
        sems = (sem0, sem1)

        def prep(c, k):
            idxc = idx_v[pl.ds((c // 4) * _LANES, _LANES)]
            for j in range(CB):
                bj = _bcast_dyn(idxc, (c % 4) * CB + j)
                isub_v[k, pl.ds(j * M, M)] = bj * M + lane
            pltpu.async_copy(bank_hbm.at[isub_v.at[k]], rows_v.at[k], sems[k])
            pltpu.async_copy(q2_hbm.at[pl.ds(base + c * CB, CB)],
                             q2_v.at[k], sems[k])

        def drain(k):
            pltpu.make_async_copy(
                bank_hbm.at[pl.ds(0, CB * M)], rows_v.at[k], sems[k]).wait()
            pltpu.make_async_copy(
                q2_hbm.at[pl.ds(0, CB)], q2_v.at[k], sems[k]).wait()

        def compute_b(k, j):
            jrow = j * M

            def p1(dc, accs):
                accs = list(accs)
                q2c = q2_v[k, j, pl.ds(dc * _LANES, _LANES)]
                for m in range(M):
                    g = rows_v[k, jrow + m, pl.ds(dc * _LANES, _LANES)]
                    accs[m] = accs[m] + g * q2c
                return tuple(accs)

            accs = lax.fori_loop(
                0, DC, p1, tuple(jnp.zeros((_LANES,), jnp.float32)
                                 for _ in range(M)))
            vs = list(accs)
            for bit in (1, 2, 4, 8):
                ix = lane ^ bit
                mk = (lane & bit) == 0
                vs = [jnp.where(mk, x + _shuf(x, ix), y + _shuf(y, ix))
                      for x, y in zip(vs[0::2], vs[1::2])]
            s = vs[0]
            mx = s
            for bit in (1, 2, 4, 8):
                mx = jnp.maximum(mx, _shuf(mx, lane ^ bit))
            e = jnp.exp(s - mx)
            z = e
            for bit in (1, 2, 4, 8):
                z = z + _shuf(z, lane ^ bit)
            a = e / z
            ab = [_bcast(a, m) for m in range(M)]

            def p2(dc, carry):
                acc = [jnp.zeros((_LANES,), jnp.float32) for _ in range(4)]
                for m in range(M):
                    acc[m % 4] = acc[m % 4] + (
                        rows_v[k, jrow + m, pl.ds(dc * _LANES, _LANES)]
                        * ab[m])
                p_v[k, j, pl.ds(dc * _LANES, _LANES)] = (
                    (acc[0] + acc[1]) + (acc[2] + acc[3]))
                return carry

            lax.fori_loop(0, DC, p2, 0)

        prep(0, 0)

        def pair(c2, carry):
            c0 = c2 * 2
            for k in (0, 1):
                c = c0 + k

                @pl.when(c + 1 < nch)
                def _():
                    prep(c + 1, 1 - k)

                drain(k)
                for j in range(CB):
                    compute_b(k, j)
                pltpu.sync_copy(p_v.at[k],
                                p_hbm.at[pl.ds(base + c * CB, CB)])
            return carry

        lax.fori_loop(0, nch // 2, pair, 0)

    return sc(idx, q2, bank3)


def _tc_attend(idx, q2, bank, cur_msg, WvT, WoT, bv2, bo2, g2, b2,
               b_off, b_cnt):
    B, D = q2.shape
    N, M, _ = bank.shape
    G = 16
    steps = b_cnt // G

    def body(idx_s, *refs):
        (bank_refs, q2_ref, msg_ref, WvT_ref, WoT_ref, bv2_ref, bo2_ref,
         g_ref, b_ref) = (refs[:G], *refs[G:G + 8])
        o_ref, Wvo_s, bvo_s = refs[G + 8], refs[G + 9], refs[G + 10]

        @pl.when(pl.program_id(0) == 0)
        def _():
            Wvo_s[...] = jnp.dot(WvT_ref[...], WoT_ref[...],
                                 preferred_element_type=jnp.float32)
            bvo_s[...] = jnp.dot(bv2_ref[...], WoT_ref[...],
                                 preferred_element_type=jnp.float32) + bo2_ref[...]

        past = jnp.concatenate([r[...] for r in bank_refs], axis=0)
        q2b = q2_ref[...]
        s = jnp.sum(past * q2b[:, None, :], axis=-1)
        e = jnp.exp(s - jnp.max(s, axis=-1, keepdims=True))
        a = e / jnp.sum(e, axis=-1, keepdims=True)
        pb = jnp.sum(past * a[:, :, None], axis=1)
        x = msg_ref[...] + jnp.dot(pb.astype(jnp.bfloat16),
                                   Wvo_s[...].astype(jnp.bfloat16),
                                   preferred_element_type=jnp.float32) + bvo_s[...]
        mu = jnp.mean(x, axis=1, keepdims=True)
        xc = x - mu
        var = jnp.mean(xc * xc, axis=1, keepdims=True)
        o_ref[...] = xc * lax.rsqrt(var + 1e-5) * g_ref[...] + b_ref[...]

    def bank_map(j):
        return lambda i, idx_s: (idx_s[b_off + i * G + j], 0, 0)

    row_map = lambda i, idx_s: (b_off // G + i, 0)
    full = lambda i, idx_s: (0, 0)
    grid_spec = pltpu.PrefetchScalarGridSpec(
        num_scalar_prefetch=1,
        grid=(steps,),
        in_specs=[pl.BlockSpec((1, M, D), bank_map(j)) for j in range(G)]
        + [
            pl.BlockSpec((G, D), row_map),
            pl.BlockSpec((G, D), row_map),
            pl.BlockSpec((D, D), full),
            pl.BlockSpec((D, D), full),
            pl.BlockSpec((1, D), full),
            pl.BlockSpec((1, D), full),
            pl.BlockSpec((1, D), full),
            pl.BlockSpec((1, D), full),
        ],
        out_specs=pl.BlockSpec((G, D), lambda i, idx_s: (i, 0)),
        scratch_shapes=[pltpu.VMEM((D, D), jnp.float32),
                        pltpu.VMEM((1, D), jnp.float32)],
    )
    return pl.pallas_call(
        body,
        grid_spec=grid_spec,
        out_shape=jax.ShapeDtypeStruct((b_cnt, D), jnp.float32),
    )(idx, *([bank] * G), q2, cur_msg, WvT, WoT, bv2, bo2, g2, b2)


def _stage_c(p, cur_msg, WvT, WoT, bv2, bo2, g2, b2):
    B, D = p.shape
    BM = 256

    def body(p_ref, msg_ref, WvT_ref, WoT_ref, bv2_ref, bo2_ref,
             g_ref, b_ref, o_ref, Wvo_s, bvo_s):
        @pl.when(pl.program_id(0) == 0)
        def _():
            Wvo_s[...] = jnp.dot(WvT_ref[...], WoT_ref[...],
                                 preferred_element_type=jnp.float32)
            bvo_s[...] = jnp.dot(bv2_ref[...], WoT_ref[...],
                                 preferred_element_type=jnp.float32) + bo2_ref[...]

        x = msg_ref[...] + jnp.dot(p_ref[...].astype(jnp.bfloat16),
                                   Wvo_s[...].astype(jnp.bfloat16),
                                   preferred_element_type=jnp.float32) + bvo_s[...]
        mu = jnp.mean(x, axis=1, keepdims=True)
        xc = x - mu
        var = jnp.mean(xc * xc, axis=1, keepdims=True)
        o_ref[...] = xc * lax.rsqrt(var + 1e-5) * g_ref[...] + b_ref[...]

    full = lambda i: (0, 0)
    blk = lambda i: (i, 0)
    return pl.pallas_call(
        body,
        grid=(B // BM,),
        in_specs=[
            pl.BlockSpec((BM, D), blk),
            pl.BlockSpec((BM, D), blk),
            pl.BlockSpec((D, D), full),
            pl.BlockSpec((D, D), full),
            pl.BlockSpec((1, D), full),
            pl.BlockSpec((1, D), full),
            pl.BlockSpec((1, D), full),
            pl.BlockSpec((1, D), full),
        ],
        out_specs=pl.BlockSpec((BM, D), blk),
        out_shape=jax.ShapeDtypeStruct((B, D), jnp.float32),
        scratch_shapes=[pltpu.VMEM((D, D), jnp.float32),
                        pltpu.VMEM((1, D), jnp.float32)],
    )(p, cur_msg, WvT, WoT, bv2, bo2, g2, b2)


def kernel(idx, cur_msg, bank, Wq, bq, Wk, bk, Wv, bv, Wo, bo, gamma, beta):
    B, D = cur_msg.shape
    N, M, _ = bank.shape
    b_tc = 2048
    b_sc = B - b_tc
    WvT, WoT = Wv.T, Wo.T
    bv2, bo2 = bv.reshape(1, D), bo.reshape(1, D)
    g2, b2 = gamma.reshape(1, D), beta.reshape(1, D)
    q2 = _stage_a(cur_msg, Wq.T, Wk, bq.reshape(1, D))
    p_sc = _stage_b(idx, q2, bank.reshape(N * M, D), M, b_sc)
    out_tc = _tc_attend(idx, q2, bank, cur_msg, WvT, WoT, bv2, bo2, g2, b2,
                        b_sc, b_tc)
    out_sc = _stage_c(p_sc, cur_msg, WvT, WoT, bv2, bo2, g2, b2)
    return jnp.concatenate([out_sc, out_tc], axis=0)


# speedup vs baseline: 1.0436x; 1.0436x over previous
import functools

import jax
import jax.numpy as jnp
from jax import lax
from jax.experimental import pallas as pl
from jax.experimental.pallas import tpu as pltpu
from jax.experimental.pallas import tpu_sc as plsc

_LANES = 16


def _bcast(v, t):
    dn = lax.GatherDimensionNumbers(
        offset_dims=(), collapsed_slice_dims=(0,), start_index_map=(0,))
    return lax.gather(v, jnp.full((_LANES, 1), t, jnp.int32), dn, (1,),
                      mode=lax.GatherScatterMode.PROMISE_IN_BOUNDS)


def _stage_a(cur_msg, WqT, Wk, bq2):
    B, D = cur_msg.shape
    BM = 256
    scale = D ** -0.5

    def body(msg_ref, WqT_ref, Wk_ref, bq2_ref, q2_ref, Wqk_s, bqk_s):
        @pl.when(pl.program_id(0) == 0)
        def _():
            Wqk_s[...] = jnp.dot(WqT_ref[...], Wk_ref[...],
                                 preferred_element_type=jnp.float32) * scale
            bqk_s[...] = jnp.dot(bq2_ref[...], Wk_ref[...],
                                 preferred_element_type=jnp.float32) * scale

        q2_ref[...] = jnp.dot(msg_ref[...].astype(jnp.bfloat16),
                              Wqk_s[...].astype(jnp.bfloat16),
                              preferred_element_type=jnp.float32) + bqk_s[...]

    return pl.pallas_call(
        body,
        grid=(B // BM,),
        in_specs=[
            pl.BlockSpec((BM, D), lambda i: (i, 0)),
            pl.BlockSpec((D, D), lambda i: (0, 0)),
            pl.BlockSpec((D, D), lambda i: (0, 0)),
            pl.BlockSpec((1, D), lambda i: (0, 0)),
        ],
        out_specs=pl.BlockSpec((BM, D), lambda i: (i, 0)),
        out_shape=jax.ShapeDtypeStruct((B, D), jnp.float32),
        scratch_shapes=[pltpu.VMEM((D, D), jnp.float32),
                        pltpu.VMEM((1, D), jnp.float32)],
    )(cur_msg, WqT, Wk, bq2)


def _bcast_dyn(v, t):
    dn = lax.GatherDimensionNumbers(
        offset_dims=(), collapsed_slice_dims=(0,), start_index_map=(0,))
    return lax.gather(v, jnp.full((_LANES, 1), 1, jnp.int32) * t, dn, (1,),
                      mode=lax.GatherScatterMode.PROMISE_IN_BOUNDS)


def _shuf(v, ix):
    dn = lax.GatherDimensionNumbers(
        offset_dims=(), collapsed_slice_dims=(0,), start_index_map=(0,))
    return lax.gather(v, ix.reshape(_LANES, 1), dn, (1,),
                      mode=lax.GatherScatterMode.PROMISE_IN_BOUNDS)


def _stage_b(idx, q2, bank3, M, b_sc):
    B, D = q2.shape
    NW = 32
    bw = b_sc // NW
    CB = 4
    nch = bw // CB
    DC = D // _LANES

    mesh = plsc.VectorSubcoreMesh(core_axis_name="c", subcore_axis_name="s")

    @functools.partial(
        pl.kernel, mesh=mesh,
        out_type=jax.ShapeDtypeStruct((b_sc, D), jnp.float32),
        compiler_params=pltpu.CompilerParams(needs_layout_passes=False),
        scratch_types=[
            pltpu.VMEM((bw,), jnp.int32),
            pltpu.VMEM((2, CB * M), jnp.int32),
            pltpu.VMEM((2, CB * M, D), jnp.float32),
            pltpu.VMEM((2, CB, D), jnp.float32),
            pltpu.VMEM((2, CB, D), jnp.float32),
            pltpu.SemaphoreType.DMA,
            pltpu.SemaphoreType.DMA,
            pltpu.SemaphoreType.DMA,
            pltpu.SemaphoreType.DMA,
        ])
    def sc(idx_hbm, q2_hbm, bank_hbm, p_hbm,
           idx_v, isub_v, rows_v, q2_v, p_v, sem0, sem1, wsem0, wsem1):
        wid = lax.axis_index("s") * 2 + lax.axis_index("c")
        base = wid * bw
        pltpu.sync_copy(idx_hbm.at[pl.ds(base, bw)], idx_v)
        lane = lax.broadcasted_iota(jnp.int32, (_LANES,), 0)
        sems = (sem0, sem1)
        wsems = (wsem0, wsem1)

        def wdrain(k):
            pltpu.make_async_copy(
                q2_hbm.at[pl.ds(0, CB)], p_v.at[k], wsems[k]).wait()

        def prep(c, k):
            idxc = idx_v[pl.ds((c // 4) * _LANES, _LANES)]
            for j in range(CB):
                bj = _bcast_dyn(idxc, (c % 4) * CB + j)
                isub_v[k, pl.ds(j * M, M)] = bj * M + lane
            pltpu.async_copy(bank_hbm.at[isub_v.at[k]], rows_v.at[k], sems[k])
            pltpu.async_copy(q2_hbm.at[pl.ds(base + c * CB, CB)],
                             q2_v.at[k], sems[k])

        def drain(k):
            pltpu.make_async_copy(
                bank_hbm.at[pl.ds(0, CB * M)], rows_v.at[k], sems[k]).wait()
            pltpu.make_async_copy(
                q2_hbm.at[pl.ds(0, CB)], q2_v.at[k], sems[k]).wait()

        def compute_b(k, j):
            jrow = j * M

            def p1(dc, accs):
                accs = list(accs)
                q2c = q2_v[k, j, pl.ds(dc * _LANES, _LANES)]
                for m in range(M):
                    g = rows_v[k, jrow + m, pl.ds(dc * _LANES, _LANES)]
                    accs[m] = accs[m] + g * q2c
                return tuple(accs)

            accs = lax.fori_loop(
                0, DC, p1, tuple(jnp.zeros((_LANES,), jnp.float32)
                                 for _ in range(M)))
            s = jnp.zeros((_LANES,), jnp.float32)
            for m in range(M):
                s = jnp.where(lane == m, jnp.sum(accs[m]), s)
            e = jnp.exp(s - jnp.max(s))
            a = e / jnp.sum(e)
            ab = [_bcast(a, m) for m in range(M)]

            def p2(dc, carry):
                acc = [jnp.zeros((_LANES,), jnp.float32) for _ in range(4)]
                for m in range(M):
                    acc[m % 4] = acc[m % 4] + (
                        rows_v[k, jrow + m, pl.ds(dc * _LANES, _LANES)]
                        * ab[m])
                p_v[k, j, pl.ds(dc * _LANES, _LANES)] = (
                    (acc[0] + acc[1]) + (acc[2] + acc[3]))
                return carry

            lax.fori_loop(0, DC, p2, 0)

        prep(0, 0)

        def pair(c2, carry):
            c0 = c2 * 2
            for k in (0, 1):
                c = c0 + k

                @pl.when(c + 1 < nch)
                def _():
                    prep(c + 1, 1 - k)

                drain(k)

                @pl.when(c >= 2)
                def _():
                    wdrain(k)

                for j in range(CB):
                    compute_b(k, j)
                pltpu.async_copy(p_v.at[k],
                                 p_hbm.at[pl.ds(base + c * CB, CB)],
                                 wsems[k])
            return carry

        lax.fori_loop(0, nch // 2, pair, 0)
        wdrain(0)
        wdrain(1)

    return sc(idx, q2, bank3)


def _tc_attend(idx, q2, bank, cur_msg, WvT, WoT, bv2, bo2, g2, b2,
               b_off, b_cnt):
    B, D = q2.shape
    N, M, _ = bank.shape
    G = 16
    steps = b_cnt // G

    def body(idx_s, *refs):
        (bank_refs, q2_ref, msg_ref, WvT_ref, WoT_ref, bv2_ref, bo2_ref,
         g_ref, b_ref) = (refs[:G], *refs[G:G + 8])
        o_ref, Wvo_s, bvo_s = refs[G + 8], refs[G + 9], refs[G + 10]

        @pl.when(pl.program_id(0) == 0)
        def _():
            Wvo_s[...] = jnp.dot(WvT_ref[...], WoT_ref[...],
                                 preferred_element_type=jnp.float32)
            bvo_s[...] = jnp.dot(bv2_ref[...], WoT_ref[...],
                                 preferred_element_type=jnp.float32) + bo2_ref[...]

        past = jnp.concatenate([r[...] for r in bank_refs], axis=0)
        q2b = q2_ref[...]
        s = jnp.sum(past * q2b[:, None, :], axis=-1)
        e = jnp.exp(s - jnp.max(s, axis=-1, keepdims=True))
        a = e / jnp.sum(e, axis=-1, keepdims=True)
        pb = jnp.sum(past * a[:, :, None], axis=1)
        x = msg_ref[...] + jnp.dot(pb.astype(jnp.bfloat16),
                                   Wvo_s[...].astype(jnp.bfloat16),
                                   preferred_element_type=jnp.float32) + bvo_s[...]
        mu = jnp.mean(x, axis=1, keepdims=True)
        xc = x - mu
        var = jnp.mean(xc * xc, axis=1, keepdims=True)
        o_ref[...] = xc * lax.rsqrt(var + 1e-5) * g_ref[...] + b_ref[...]

    def bank_map(j):
        return lambda i, idx_s: (idx_s[b_off + i * G + j], 0, 0)

    row_map = lambda i, idx_s: (b_off // G + i, 0)
    full = lambda i, idx_s: (0, 0)
    grid_spec = pltpu.PrefetchScalarGridSpec(
        num_scalar_prefetch=1,
        grid=(steps,),
        in_specs=[pl.BlockSpec((1, M, D), bank_map(j)) for j in range(G)]
        + [
            pl.BlockSpec((G, D), row_map),
            pl.BlockSpec((G, D), row_map),
            pl.BlockSpec((D, D), full),
            pl.BlockSpec((D, D), full),
            pl.BlockSpec((1, D), full),
            pl.BlockSpec((1, D), full),
            pl.BlockSpec((1, D), full),
            pl.BlockSpec((1, D), full),
        ],
        out_specs=pl.BlockSpec((G, D), lambda i, idx_s: (i, 0)),
        scratch_shapes=[pltpu.VMEM((D, D), jnp.float32),
                        pltpu.VMEM((1, D), jnp.float32)],
    )
    return pl.pallas_call(
        body,
        grid_spec=grid_spec,
        out_shape=jax.ShapeDtypeStruct((b_cnt, D), jnp.float32),
    )(idx, *([bank] * G), q2, cur_msg, WvT, WoT, bv2, bo2, g2, b2)


def _stage_c(p, cur_msg, WvT, WoT, bv2, bo2, g2, b2):
    B, D = p.shape
    BM = 256

    def body(p_ref, msg_ref, WvT_ref, WoT_ref, bv2_ref, bo2_ref,
             g_ref, b_ref, o_ref, Wvo_s, bvo_s):
        @pl.when(pl.program_id(0) == 0)
        def _():
            Wvo_s[...] = jnp.dot(WvT_ref[...], WoT_ref[...],
                                 preferred_element_type=jnp.float32)
            bvo_s[...] = jnp.dot(bv2_ref[...], WoT_ref[...],
                                 preferred_element_type=jnp.float32) + bo2_ref[...]

        x = msg_ref[...] + jnp.dot(p_ref[...].astype(jnp.bfloat16),
                                   Wvo_s[...].astype(jnp.bfloat16),
                                   preferred_element_type=jnp.float32) + bvo_s[...]
        mu = jnp.mean(x, axis=1, keepdims=True)
        xc = x - mu
        var = jnp.mean(xc * xc, axis=1, keepdims=True)
        o_ref[...] = xc * lax.rsqrt(var + 1e-5) * g_ref[...] + b_ref[...]

    full = lambda i: (0, 0)
    blk = lambda i: (i, 0)
    return pl.pallas_call(
        body,
        grid=(B // BM,),
        in_specs=[
            pl.BlockSpec((BM, D), blk),
            pl.BlockSpec((BM, D), blk),
            pl.BlockSpec((D, D), full),
            pl.BlockSpec((D, D), full),
            pl.BlockSpec((1, D), full),
            pl.BlockSpec((1, D), full),
            pl.BlockSpec((1, D), full),
            pl.BlockSpec((1, D), full),
        ],
        out_specs=pl.BlockSpec((BM, D), blk),
        out_shape=jax.ShapeDtypeStruct((B, D), jnp.float32),
        scratch_shapes=[pltpu.VMEM((D, D), jnp.float32),
                        pltpu.VMEM((1, D), jnp.float32)],
    )(p, cur_msg, WvT, WoT, bv2, bo2, g2, b2)


def kernel(idx, cur_msg, bank, Wq, bq, Wk, bk, Wv, bv, Wo, bo, gamma, beta):
    B, D = cur_msg.shape
    N, M, _ = bank.shape
    b_tc = 2048
    b_sc = B - b_tc
    WvT, WoT = Wv.T, Wo.T
    bv2, bo2 = bv.reshape(1, D), bo.reshape(1, D)
    g2, b2 = gamma.reshape(1, D), beta.reshape(1, D)
    q2 = _stage_a(cur_msg, Wq.T, Wk, bq.reshape(1, D))
    p_sc = _stage_b(idx, q2, bank.reshape(N * M, D), M, b_sc)
    out_tc = _tc_attend(idx, q2, bank, cur_msg, WvT, WoT, bv2, bo2, g2, b2,
                        b_sc, b_tc)
    out_sc = _stage_c(p_sc, cur_msg, WvT, WoT, bv2, bo2, g2, b2)
    return jnp.concatenate([out_sc, out_tc], axis=0)


# speedup vs baseline: 1.1071x; 1.0608x over previous
import functools

import jax
import jax.numpy as jnp
from jax import lax
from jax.experimental import pallas as pl
from jax.experimental.pallas import tpu as pltpu
from jax.experimental.pallas import tpu_sc as plsc

_LANES = 16


def _bcast(v, t):
    dn = lax.GatherDimensionNumbers(
        offset_dims=(), collapsed_slice_dims=(0,), start_index_map=(0,))
    return lax.gather(v, jnp.full((_LANES, 1), t, jnp.int32), dn, (1,),
                      mode=lax.GatherScatterMode.PROMISE_IN_BOUNDS)


def _stage_a(cur_msg, WqT, Wk, bq2):
    B, D = cur_msg.shape
    BM = 512
    scale = D ** -0.5

    def body(msg_ref, WqT_ref, Wk_ref, bq2_ref, q2_ref, Wqk_s, bqk_s):
        @pl.when(pl.program_id(0) == 0)
        def _():
            Wqk_s[...] = jnp.dot(WqT_ref[...], Wk_ref[...],
                                 preferred_element_type=jnp.float32) * scale
            bqk_s[...] = jnp.dot(bq2_ref[...], Wk_ref[...],
                                 preferred_element_type=jnp.float32) * scale

        q2_ref[...] = jnp.dot(msg_ref[...].astype(jnp.bfloat16),
                              Wqk_s[...].astype(jnp.bfloat16),
                              preferred_element_type=jnp.float32) + bqk_s[...]

    return pl.pallas_call(
        body,
        grid=(B // BM,),
        in_specs=[
            pl.BlockSpec((BM, D), lambda i: (i, 0)),
            pl.BlockSpec((D, D), lambda i: (0, 0)),
            pl.BlockSpec((D, D), lambda i: (0, 0)),
            pl.BlockSpec((1, D), lambda i: (0, 0)),
        ],
        out_specs=pl.BlockSpec((BM, D), lambda i: (i, 0)),
        out_shape=jax.ShapeDtypeStruct((B, D), jnp.float32),
        scratch_shapes=[pltpu.VMEM((D, D), jnp.float32),
                        pltpu.VMEM((1, D), jnp.float32)],
    )(cur_msg, WqT, Wk, bq2)


def _bcast_dyn(v, t):
    dn = lax.GatherDimensionNumbers(
        offset_dims=(), collapsed_slice_dims=(0,), start_index_map=(0,))
    return lax.gather(v, jnp.full((_LANES, 1), 1, jnp.int32) * t, dn, (1,),
                      mode=lax.GatherScatterMode.PROMISE_IN_BOUNDS)


def _shuf(v, ix):
    dn = lax.GatherDimensionNumbers(
        offset_dims=(), collapsed_slice_dims=(0,), start_index_map=(0,))
    return lax.gather(v, ix.reshape(_LANES, 1), dn, (1,),
                      mode=lax.GatherScatterMode.PROMISE_IN_BOUNDS)


def _stage_b(idx, q2, bank3, M, b_sc):
    B, D = q2.shape
    NW = 32
    bw = b_sc // NW
    CB = 4
    nch = bw // CB
    DC = D // _LANES

    mesh = plsc.VectorSubcoreMesh(core_axis_name="c", subcore_axis_name="s")

    @functools.partial(
        pl.kernel, mesh=mesh,
        out_type=jax.ShapeDtypeStruct((b_sc, D), jnp.float32),
        compiler_params=pltpu.CompilerParams(needs_layout_passes=False),
        scratch_types=[
            pltpu.VMEM((bw,), jnp.int32),
            pltpu.VMEM((2, CB * M), jnp.int32),
            pltpu.VMEM((2, CB * M, D), jnp.float32),
            pltpu.VMEM((2, CB, D), jnp.float32),
            pltpu.VMEM((2, CB, D), jnp.float32),
            pltpu.SemaphoreType.DMA,
            pltpu.SemaphoreType.DMA,
            pltpu.SemaphoreType.DMA,
            pltpu.SemaphoreType.DMA,
        ])
    def sc(idx_hbm, q2_hbm, bank_hbm, p_hbm,
           idx_v, isub_v, rows_v, q2_v, p_v, sem0, sem1, wsem0, wsem1):
        wid = lax.axis_index("s") * 2 + lax.axis_index("c")
        base = wid * bw
        pltpu.sync_copy(idx_hbm.at[pl.ds(base, bw)], idx_v)
        lane = lax.broadcasted_iota(jnp.int32, (_LANES,), 0)
        sems = (sem0, sem1)
        wsems = (wsem0, wsem1)

        def wdrain(k):
            pltpu.make_async_copy(
                q2_hbm.at[pl.ds(0, CB)], p_v.at[k], wsems[k]).wait()

        def prep(c, k):
            idxc = idx_v[pl.ds((c // 4) * _LANES, _LANES)]
            for j in range(CB):
                bj = _bcast_dyn(idxc, (c % 4) * CB + j)
                isub_v[k, pl.ds(j * M, M)] = bj * M + lane
            pltpu.async_copy(bank_hbm.at[isub_v.at[k]], rows_v.at[k], sems[k])
            pltpu.async_copy(q2_hbm.at[pl.ds(base + c * CB, CB)],
                             q2_v.at[k], sems[k])

        def drain(k):
            pltpu.make_async_copy(
                bank_hbm.at[pl.ds(0, CB * M)], rows_v.at[k], sems[k]).wait()
            pltpu.make_async_copy(
                q2_hbm.at[pl.ds(0, CB)], q2_v.at[k], sems[k]).wait()

        def compute_b(k, j):
            jrow = j * M

            def p1(dc, accs):
                accs = list(accs)
                q2c = q2_v[k, j, pl.ds(dc * _LANES, _LANES)]
                for m in range(M):
                    g = rows_v[k, jrow + m, pl.ds(dc * _LANES, _LANES)]
                    accs[m] = accs[m] + g * q2c
                return tuple(accs)

            accs = lax.fori_loop(
                0, DC, p1, tuple(jnp.zeros((_LANES,), jnp.float32)
                                 for _ in range(M)))
            s = jnp.zeros((_LANES,), jnp.float32)
            for m in range(M):
                s = jnp.where(lane == m, jnp.sum(accs[m]), s)
            e = jnp.exp(s - jnp.max(s))
            a = e / jnp.sum(e)
            ab = [_bcast(a, m) for m in range(M)]

            def p2(dc, carry):
                acc = [jnp.zeros((_LANES,), jnp.float32) for _ in range(4)]
                for m in range(M):
                    acc[m % 4] = acc[m % 4] + (
                        rows_v[k, jrow + m, pl.ds(dc * _LANES, _LANES)]
                        * ab[m])
                p_v[k, j, pl.ds(dc * _LANES, _LANES)] = (
                    (acc[0] + acc[1]) + (acc[2] + acc[3]))
                return carry

            lax.fori_loop(0, DC, p2, 0)

        prep(0, 0)

        def pair(c2, carry):
            c0 = c2 * 2
            for k in (0, 1):
                c = c0 + k

                @pl.when(c + 1 < nch)
                def _():
                    prep(c + 1, 1 - k)

                drain(k)

                @pl.when(c >= 2)
                def _():
                    wdrain(k)

                for j in range(CB):
                    compute_b(k, j)
                pltpu.async_copy(p_v.at[k],
                                 p_hbm.at[pl.ds(base + c * CB, CB)],
                                 wsems[k])
            return carry

        lax.fori_loop(0, nch // 2, pair, 0)
        wdrain(0)
        wdrain(1)

    return sc(idx, q2, bank3)


def _tc_attend(idx, q2, bank, cur_msg, WvT, WoT, bv2, bo2, g2, b2,
               b_off, b_cnt):
    B, D = q2.shape
    N, M, _ = bank.shape
    G = 16
    steps = b_cnt // G

    def body(idx_s, *refs):
        (bank_refs, q2_ref, msg_ref, WvT_ref, WoT_ref, bv2_ref, bo2_ref,
         g_ref, b_ref) = (refs[:G], *refs[G:G + 8])
        o_ref, Wvo_s, bvo_s = refs[G + 8], refs[G + 9], refs[G + 10]

        @pl.when(pl.program_id(0) == 0)
        def _():
            Wvo_s[...] = jnp.dot(WvT_ref[...], WoT_ref[...],
                                 preferred_element_type=jnp.float32)
            bvo_s[...] = jnp.dot(bv2_ref[...], WoT_ref[...],
                                 preferred_element_type=jnp.float32) + bo2_ref[...]

        past = jnp.concatenate([r[...] for r in bank_refs], axis=0)
        q2b = q2_ref[...]
        s = jnp.sum(past * q2b[:, None, :], axis=-1)
        e = jnp.exp(s - jnp.max(s, axis=-1, keepdims=True))
        a = e / jnp.sum(e, axis=-1, keepdims=True)
        pb = jnp.sum(past * a[:, :, None], axis=1)
        x = msg_ref[...] + jnp.dot(pb.astype(jnp.bfloat16),
                                   Wvo_s[...].astype(jnp.bfloat16),
                                   preferred_element_type=jnp.float32) + bvo_s[...]
        mu = jnp.mean(x, axis=1, keepdims=True)
        xc = x - mu
        var = jnp.mean(xc * xc, axis=1, keepdims=True)
        o_ref[...] = xc * lax.rsqrt(var + 1e-5) * g_ref[...] + b_ref[...]

    def bank_map(j):
        return lambda i, idx_s: (idx_s[b_off + i * G + j], 0, 0)

    row_map = lambda i, idx_s: (b_off // G + i, 0)
    full = lambda i, idx_s: (0, 0)
    grid_spec = pltpu.PrefetchScalarGridSpec(
        num_scalar_prefetch=1,
        grid=(steps,),
        in_specs=[pl.BlockSpec((1, M, D), bank_map(j)) for j in range(G)]
        + [
            pl.BlockSpec((G, D), row_map),
            pl.BlockSpec((G, D), row_map),
            pl.BlockSpec((D, D), full),
            pl.BlockSpec((D, D), full),
            pl.BlockSpec((1, D), full),
            pl.BlockSpec((1, D), full),
            pl.BlockSpec((1, D), full),
            pl.BlockSpec((1, D), full),
        ],
        out_specs=pl.BlockSpec((G, D), lambda i, idx_s: (i, 0)),
        scratch_shapes=[pltpu.VMEM((D, D), jnp.float32),
                        pltpu.VMEM((1, D), jnp.float32)],
    )
    return pl.pallas_call(
        body,
        grid_spec=grid_spec,
        out_shape=jax.ShapeDtypeStruct((b_cnt, D), jnp.float32),
    )(idx, *([bank] * G), q2, cur_msg, WvT, WoT, bv2, bo2, g2, b2)


def _stage_c(p, cur_msg, WvT, WoT, bv2, bo2, g2, b2):
    B, D = p.shape
    BM = 512

    def body(p_ref, msg_ref, WvT_ref, WoT_ref, bv2_ref, bo2_ref,
             g_ref, b_ref, o_ref, Wvo_s, bvo_s):
        @pl.when(pl.program_id(0) == 0)
        def _():
            Wvo_s[...] = jnp.dot(WvT_ref[...], WoT_ref[...],
                                 preferred_element_type=jnp.float32)
            bvo_s[...] = jnp.dot(bv2_ref[...], WoT_ref[...],
                                 preferred_element_type=jnp.float32) + bo2_ref[...]

        x = msg_ref[...] + jnp.dot(p_ref[...].astype(jnp.bfloat16),
                                   Wvo_s[...].astype(jnp.bfloat16),
                                   preferred_element_type=jnp.float32) + bvo_s[...]
        mu = jnp.mean(x, axis=1, keepdims=True)
        xc = x - mu
        var = jnp.mean(xc * xc, axis=1, keepdims=True)
        o_ref[...] = xc * lax.rsqrt(var + 1e-5) * g_ref[...] + b_ref[...]

    full = lambda i: (0, 0)
    blk = lambda i: (i, 0)
    return pl.pallas_call(
        body,
        grid=(B // BM,),
        in_specs=[
            pl.BlockSpec((BM, D), blk),
            pl.BlockSpec((BM, D), blk),
            pl.BlockSpec((D, D), full),
            pl.BlockSpec((D, D), full),
            pl.BlockSpec((1, D), full),
            pl.BlockSpec((1, D), full),
            pl.BlockSpec((1, D), full),
            pl.BlockSpec((1, D), full),
        ],
        out_specs=pl.BlockSpec((BM, D), blk),
        out_shape=jax.ShapeDtypeStruct((B, D), jnp.float32),
        scratch_shapes=[pltpu.VMEM((D, D), jnp.float32),
                        pltpu.VMEM((1, D), jnp.float32)],
    )(p, cur_msg, WvT, WoT, bv2, bo2, g2, b2)


def kernel(idx, cur_msg, bank, Wq, bq, Wk, bk, Wv, bv, Wo, bo, gamma, beta):
    B, D = cur_msg.shape
    N, M, _ = bank.shape
    b_tc = 2048
    b_sc = B - b_tc
    WvT, WoT = Wv.T, Wo.T
    bv2, bo2 = bv.reshape(1, D), bo.reshape(1, D)
    g2, b2 = gamma.reshape(1, D), beta.reshape(1, D)
    q2 = _stage_a(cur_msg, Wq.T, Wk, bq.reshape(1, D))
    p_sc = _stage_b(idx, q2, bank.reshape(N * M, D), M, b_sc)
    out_tc = _tc_attend(idx, q2, bank, cur_msg, WvT, WoT, bv2, bo2, g2, b2,
                        b_sc, b_tc)
    out_sc = _stage_c(p_sc, cur_msg, WvT, WoT, bv2, bo2, g2, b2)
    return jnp.concatenate([out_sc, out_tc], axis=0)


# speedup vs baseline: 1.1082x; 1.0010x over previous
import functools

import jax
import jax.numpy as jnp
from jax import lax
from jax.experimental import pallas as pl
from jax.experimental.pallas import tpu as pltpu
from jax.experimental.pallas import tpu_sc as plsc

_LANES = 16


def _bcast(v, t):
    dn = lax.GatherDimensionNumbers(
        offset_dims=(), collapsed_slice_dims=(0,), start_index_map=(0,))
    return lax.gather(v, jnp.full((_LANES, 1), t, jnp.int32), dn, (1,),
                      mode=lax.GatherScatterMode.PROMISE_IN_BOUNDS)


def _stage_a(cur_msg, WqT, Wk, bq2):
    B, D = cur_msg.shape
    BM = 512
    scale = D ** -0.5

    def body(msg_ref, WqT_ref, Wk_ref, bq2_ref, q2_ref, Wqk_s, bqk_s):
        @pl.when(pl.program_id(0) == 0)
        def _():
            Wqk_s[...] = jnp.dot(WqT_ref[...], Wk_ref[...],
                                 preferred_element_type=jnp.float32) * scale
            bqk_s[...] = jnp.dot(bq2_ref[...], Wk_ref[...],
                                 preferred_element_type=jnp.float32) * scale

        q2_ref[...] = jnp.dot(msg_ref[...].astype(jnp.bfloat16),
                              Wqk_s[...].astype(jnp.bfloat16),
                              preferred_element_type=jnp.float32) + bqk_s[...]

    return pl.pallas_call(
        body,
        grid=(B // BM,),
        in_specs=[
            pl.BlockSpec((BM, D), lambda i: (i, 0)),
            pl.BlockSpec((D, D), lambda i: (0, 0)),
            pl.BlockSpec((D, D), lambda i: (0, 0)),
            pl.BlockSpec((1, D), lambda i: (0, 0)),
        ],
        out_specs=pl.BlockSpec((BM, D), lambda i: (i, 0)),
        out_shape=jax.ShapeDtypeStruct((B, D), jnp.float32),
        scratch_shapes=[pltpu.VMEM((D, D), jnp.float32),
                        pltpu.VMEM((1, D), jnp.float32)],
    )(cur_msg, WqT, Wk, bq2)


def _bcast_dyn(v, t):
    dn = lax.GatherDimensionNumbers(
        offset_dims=(), collapsed_slice_dims=(0,), start_index_map=(0,))
    return lax.gather(v, jnp.full((_LANES, 1), 1, jnp.int32) * t, dn, (1,),
                      mode=lax.GatherScatterMode.PROMISE_IN_BOUNDS)


def _stage_b(idx, q2, bank3, M, b_sc):
    B, D = q2.shape
    NW = 32
    bw = b_sc // NW
    CB = 4
    nch = bw // CB
    DC = D // _LANES

    mesh = plsc.VectorSubcoreMesh(core_axis_name="c", subcore_axis_name="s")

    @functools.partial(
        pl.kernel, mesh=mesh,
        out_type=jax.ShapeDtypeStruct((b_sc, D), jnp.float32),
        compiler_params=pltpu.CompilerParams(needs_layout_passes=False),
        scratch_types=[
            pltpu.VMEM((bw,), jnp.int32),
            pltpu.VMEM((2, CB * M), jnp.int32),
            pltpu.VMEM((2, CB * M, D), jnp.float32),
            pltpu.VMEM((2, CB, D), jnp.float32),
            pltpu.VMEM((2, CB, D), jnp.float32),
            pltpu.SemaphoreType.DMA,
            pltpu.SemaphoreType.DMA,
            pltpu.SemaphoreType.DMA,
            pltpu.SemaphoreType.DMA,
        ])
    def sc(idx_hbm, q2_hbm, bank_hbm, p_hbm,
           idx_v, isub_v, rows_v, q2_v, p_v, sem0, sem1, wsem0, wsem1):
        wid = lax.axis_index("s") * 2 + lax.axis_index("c")
        base = wid * bw
        pltpu.sync_copy(idx_hbm.at[pl.ds(base, bw)], idx_v)
        lane = lax.broadcasted_iota(jnp.int32, (_LANES,), 0)
        sems = (sem0, sem1)
        wsems = (wsem0, wsem1)

        def wdrain(k):
            pltpu.make_async_copy(
                q2_hbm.at[pl.ds(0, CB)], p_v.at[k], wsems[k]).wait()

        def prep(c, k):
            idxc = idx_v[pl.ds((c // 4) * _LANES, _LANES)]
            for j in range(CB):
                bj = _bcast_dyn(idxc, (c % 4) * CB + j)
                isub_v[k, pl.ds(j * M, M)] = bj * M + lane
            pltpu.async_copy(bank_hbm.at[isub_v.at[k]], rows_v.at[k], sems[k])
            pltpu.async_copy(q2_hbm.at[pl.ds(base + c * CB, CB)],
                             q2_v.at[k], sems[k])

        def drain(k):
            pltpu.make_async_copy(
                bank_hbm.at[pl.ds(0, CB * M)], rows_v.at[k], sems[k]).wait()
            pltpu.make_async_copy(
                q2_hbm.at[pl.ds(0, CB)], q2_v.at[k], sems[k]).wait()

        def compute_b(k, j):
            jrow = j * M

            def p1(dc, accs):
                accs = list(accs)
                q2c = q2_v[k, j, pl.ds(dc * _LANES, _LANES)]
                for m in range(M):
                    g = rows_v[k, jrow + m, pl.ds(dc * _LANES, _LANES)]
                    accs[m] = accs[m] + g * q2c
                return tuple(accs)

            accs = lax.fori_loop(
                0, DC, p1, tuple(jnp.zeros((_LANES,), jnp.float32)
                                 for _ in range(M)))
            s = jnp.zeros((_LANES,), jnp.float32)
            for m in range(M):
                s = jnp.where(lane == m, jnp.sum(accs[m]), s)
            e = jnp.exp(s - jnp.max(s))
            a = e / jnp.sum(e)
            ab = [_bcast(a, m) for m in range(M)]

            def p2(dc, carry):
                acc = [jnp.zeros((_LANES,), jnp.float32) for _ in range(4)]
                for m in range(M):
                    acc[m % 4] = acc[m % 4] + (
                        rows_v[k, jrow + m, pl.ds(dc * _LANES, _LANES)]
                        * ab[m])
                p_v[k, j, pl.ds(dc * _LANES, _LANES)] = (
                    (acc[0] + acc[1]) + (acc[2] + acc[3]))
                return carry

            lax.fori_loop(0, DC, p2, 0)

        prep(0, 0)

        def pair(c2, carry):
            c0 = c2 * 2
            for k in (0, 1):
                c = c0 + k

                @pl.when(c + 1 < nch)
                def _():
                    prep(c + 1, 1 - k)

                drain(k)

                @pl.when(c >= 2)
                def _():
                    wdrain(k)

                for j in range(CB):
                    compute_b(k, j)
                pltpu.async_copy(p_v.at[k],
                                 p_hbm.at[pl.ds(base + c * CB, CB)],
                                 wsems[k])
            return carry

        lax.fori_loop(0, nch // 2, pair, 0)
        wdrain(0)
        wdrain(1)

    return sc(idx, q2, bank3)


def _tc_attend(idx, q2, bank, cur_msg, WvT, WoT, bv2, bo2, g2, b2,
               b_off, b_cnt):
    B, D = q2.shape
    N, M, _ = bank.shape
    G = 16
    steps = b_cnt // G

    def body(idx_s, *refs):
        (bank_refs, q2_ref, msg_ref, WvT_ref, WoT_ref, bv2_ref, bo2_ref,
         g_ref, b_ref) = (refs[:G], *refs[G:G + 8])
        o_ref, Wvo_s, bvo_s = refs[G + 8], refs[G + 9], refs[G + 10]

        @pl.when(pl.program_id(0) == 0)
        def _():
            Wvo_s[...] = jnp.dot(WvT_ref[...], WoT_ref[...],
                                 preferred_element_type=jnp.float32)
            bvo_s[...] = jnp.dot(bv2_ref[...], WoT_ref[...],
                                 preferred_element_type=jnp.float32) + bo2_ref[...]

        past = jnp.concatenate([r[...] for r in bank_refs], axis=0)
        q2b = q2_ref[...]
        s = jnp.sum(past * q2b[:, None, :], axis=-1)
        e = jnp.exp(s - jnp.max(s, axis=-1, keepdims=True))
        a = e / jnp.sum(e, axis=-1, keepdims=True)
        pb = jnp.sum(past * a[:, :, None], axis=1)
        x = msg_ref[...] + jnp.dot(pb.astype(jnp.bfloat16),
                                   Wvo_s[...].astype(jnp.bfloat16),
                                   preferred_element_type=jnp.float32) + bvo_s[...]
        mu = jnp.mean(x, axis=1, keepdims=True)
        xc = x - mu
        var = jnp.mean(xc * xc, axis=1, keepdims=True)
        o_ref[...] = xc * lax.rsqrt(var + 1e-5) * g_ref[...] + b_ref[...]

    def bank_map(j):
        return lambda i, idx_s: (idx_s[b_off + i * G + j], 0, 0)

    row_map = lambda i, idx_s: (b_off // G + i, 0)
    full = lambda i, idx_s: (0, 0)
    grid_spec = pltpu.PrefetchScalarGridSpec(
        num_scalar_prefetch=1,
        grid=(steps,),
        in_specs=[pl.BlockSpec((1, M, D), bank_map(j)) for j in range(G)]
        + [
            pl.BlockSpec((G, D), row_map),
            pl.BlockSpec((G, D), row_map),
            pl.BlockSpec((D, D), full),
            pl.BlockSpec((D, D), full),
            pl.BlockSpec((1, D), full),
            pl.BlockSpec((1, D), full),
            pl.BlockSpec((1, D), full),
            pl.BlockSpec((1, D), full),
        ],
        out_specs=pl.BlockSpec((G, D), lambda i, idx_s: (i, 0)),
        scratch_shapes=[pltpu.VMEM((D, D), jnp.float32),
                        pltpu.VMEM((1, D), jnp.float32)],
    )
    return pl.pallas_call(
        body,
        grid_spec=grid_spec,
        out_shape=jax.ShapeDtypeStruct((b_cnt, D), jnp.float32),
    )(idx, *([bank] * G), q2, cur_msg, WvT, WoT, bv2, bo2, g2, b2)


def _stage_c(p, cur_msg, WvT, WoT, bv2, bo2, g2, b2):
    B, D = p.shape
    BM = 512

    def body(p_ref, msg_ref, WvT_ref, WoT_ref, bv2_ref, bo2_ref,
             g_ref, b_ref, o_ref, Wvo_s, bvo_s):
        @pl.when(pl.program_id(0) == 0)
        def _():
            Wvo_s[...] = jnp.dot(WvT_ref[...], WoT_ref[...],
                                 preferred_element_type=jnp.float32)
            bvo_s[...] = jnp.dot(bv2_ref[...], WoT_ref[...],
                                 preferred_element_type=jnp.float32) + bo2_ref[...]

        x = msg_ref[...] + jnp.dot(p_ref[...].astype(jnp.bfloat16),
                                   Wvo_s[...].astype(jnp.bfloat16),
                                   preferred_element_type=jnp.float32) + bvo_s[...]
        mu = jnp.mean(x, axis=1, keepdims=True)
        xc = x - mu
        var = jnp.mean(xc * xc, axis=1, keepdims=True)
        o_ref[...] = xc * lax.rsqrt(var + 1e-5) * g_ref[...] + b_ref[...]

    full = lambda i: (0, 0)
    blk = lambda i: (i, 0)
    return pl.pallas_call(
        body,
        grid=(B // BM,),
        in_specs=[
            pl.BlockSpec((BM, D), blk),
            pl.BlockSpec((BM, D), blk),
            pl.BlockSpec((D, D), full),
            pl.BlockSpec((D, D), full),
            pl.BlockSpec((1, D), full),
            pl.BlockSpec((1, D), full),
            pl.BlockSpec((1, D), full),
            pl.BlockSpec((1, D), full),
        ],
        out_specs=pl.BlockSpec((BM, D), blk),
        out_shape=jax.ShapeDtypeStruct((B, D), jnp.float32),
        scratch_shapes=[pltpu.VMEM((D, D), jnp.float32),
                        pltpu.VMEM((1, D), jnp.float32)],
    )(p, cur_msg, WvT, WoT, bv2, bo2, g2, b2)


def kernel(idx, cur_msg, bank, Wq, bq, Wk, bk, Wv, bv, Wo, bo, gamma, beta):
    B, D = cur_msg.shape
    N, M, _ = bank.shape
    b_tc = 2048
    b_sc = B - b_tc
    WvT, WoT = Wv.T, Wo.T
    bv2, bo2 = bv.reshape(1, D), bo.reshape(1, D)
    g2, b2 = gamma.reshape(1, D), beta.reshape(1, D)
    q2 = _stage_a(cur_msg, Wq.T, Wk, bq.reshape(1, D))
    p_sc = _stage_b(idx, q2, bank.reshape(N * M, D), M, b_sc)
    out_tc = _tc_attend(idx, q2, bank, cur_msg, WvT, WoT, bv2, bo2, g2, b2,
                        b_sc, b_tc)
    out_sc = _stage_c(p_sc, cur_msg, WvT, WoT, bv2, bo2, g2, b2)
    return jnp.concatenate([out_sc, out_tc], axis=0)
